# baseline (speedup 1.0000x reference)
import functools

import jax
import jax.numpy as jnp
from jax import lax
from jax.experimental import pallas as pl
from jax.experimental.pallas import tpu as pltpu
from jax.experimental.pallas import tpu_sc as plsc


def _proj_body(emb_ref, w_ref, b_ref, out_ref):
    out_ref[...] = lax.dot_general(
        emb_ref[...], w_ref[...],
        dimension_numbers=(((1,), (1,)), ((), ())),
        preferred_element_type=jnp.float32,
    ) + b_ref[...]


def _project_table(emb_table, W1, b1):
    V, D = emb_table.shape
    return pl.pallas_call(
        _proj_body,
        out_shape=jax.ShapeDtypeStruct((V, D), jnp.float32),
    )(emb_table, W1, b1.reshape(1, D))


_P = 128
_CHUNK = 128
_NBUF = 4
_PROJ_ROWS = 4096


def _make_sc_gather(num_rows):
    info = plsc.get_sparse_core_info()
    NC, NS = info.num_cores, info.num_subcores
    NW = NC * NS
    rows_per_w = num_rows // NW
    chunks_per_w = rows_per_w // _CHUNK
    quads = chunks_per_w // _NBUF
    mesh = plsc.VectorSubcoreMesh(core_axis_name="c", subcore_axis_name="s")

    @functools.partial(
        pl.kernel,
        mesh=mesh,
        out_type=jax.ShapeDtypeStruct((num_rows, _P), jnp.float32),
        scratch_types=[
            pltpu.VMEM((chunks_per_w, _CHUNK), jnp.int32),
        ] + [pltpu.VMEM((_CHUNK, _P), jnp.float32)] * _NBUF
          + [pltpu.SemaphoreType.DMA] * (2 * _NBUF),
    )
    def sc_kernel(tab_hbm, idx_hbm, out_hbm, idx_v, *bufs_sems):
        bufs = bufs_sems[:_NBUF]
        gsems = bufs_sems[_NBUF:2 * _NBUF]
        osems = bufs_sems[2 * _NBUF:]
        scid = lax.axis_index("c")
        sid = lax.axis_index("s")
        wid = scid * NS + sid
        r0 = wid * rows_per_w

        pltpu.sync_copy(idx_hbm.at[wid], idx_v)

        def fire_gather(j, k):
            return pltpu.async_copy(tab_hbm.at[idx_v.at[j]], bufs[k],
                                    gsems[k])

        def out_slice(j):
            return out_hbm.at[pl.ds(pl.multiple_of(r0 + j * _CHUNK, _CHUNK),
                                    _CHUNK)]

        for k in range(_NBUF):
            fire_gather(k, k)

        def quad(g, last):
            for k in range(_NBUF):
                j = g * _NBUF + k
                pltpu.make_async_copy(tab_hbm.at[idx_v.at[j]], bufs[k],
                                      gsems[k]).wait()
                pltpu.async_copy(bufs[k], out_slice(j), osems[k])
            for k in range(_NBUF):
                j = g * _NBUF + k
                pltpu.make_async_copy(bufs[k], out_slice(j), osems[k]).wait()
                if not last:
                    fire_gather(j + _NBUF, k)

        lax.fori_loop(0, quads - 1, lambda g, c: (quad(g, False), c)[1], 0)
        quad(quads - 1, True)

    return sc_kernel


_RB = 512
_NSLICE = 4


def _relayout_body(in_ref, out_ref):
    out_ref[...] = in_ref[...].reshape(out_ref.shape)


def _relayout_acc_body(in_ref, acc_ref, out_ref):
    del acc_ref
    out_ref[...] = in_ref[...].reshape(out_ref.shape)


def _relayout_slice(out_flat_s, acc, s, num_tokens, D2):
    toks = out_flat_s.shape[0] * _P // D2
    blocks = toks // _RB
    ppr = D2 // _P
    out_shape = jax.ShapeDtypeStruct((num_tokens, D2), jnp.float32)
    if acc is None:
        return pl.pallas_call(
            _relayout_body,
            grid=(blocks,),
            in_specs=[pl.BlockSpec((_RB * ppr, _P), lambda i: (i, 0))],
            out_specs=pl.BlockSpec((_RB, D2),
                                   lambda i, s=s, b=blocks: (i + s * b, 0)),
            out_shape=out_shape,
        )(out_flat_s)
    return pl.pallas_call(
        _relayout_acc_body,
        grid=(blocks,),
        in_specs=[pl.BlockSpec((_RB * ppr, _P), lambda i: (i, 0)),
                  pl.BlockSpec(memory_space=pl.ANY)],
        out_specs=pl.BlockSpec((_RB, D2),
                               lambda i, s=s, b=blocks: (i + s * b, 0)),
        out_shape=out_shape,
        input_output_aliases={1: 0},
    )(out_flat_s, acc)


def kernel(caption_tknID, img_fc, emb_table, W1, b1):
    B, L = caption_tknID.shape
    D = img_fc.shape[1]
    num_tokens = B * L
    pieces = D // _P
    num_rows = num_tokens * 2 * pieces

    proj = _project_table(emb_table, W1, b1)

    tab = jnp.concatenate([
        proj.reshape(emb_table.shape[0] * pieces, _P),
        jnp.zeros((_PROJ_ROWS - emb_table.shape[0] * pieces, _P), jnp.float32),
        img_fc.reshape(B * pieces, _P),
    ], axis=0)

    info = plsc.get_sparse_core_info()
    nw = info.num_cores * info.num_subcores
    tokT = caption_tknID.astype(jnp.int32).T.reshape(num_tokens)
    u = jax.lax.broadcasted_iota(jnp.int32, (num_rows,), 0)
    l = u // (B * 2 * pieces)
    rem = u % (B * 2 * pieces)
    rb = rem // (16 * pieces)
    q = rem % (16 * pieces)
    cb = q // 8
    r = q % 8
    b = rb * 8 + r
    tok_g = jnp.take(tokT, l * B + b)
    idx_flat = jnp.where(cb < pieces,
                         tok_g * pieces + cb,
                         _PROJ_ROWS + b * pieces + (cb - pieces))
    idx3d = idx_flat.reshape(nw, num_rows // (nw * _CHUNK), _CHUNK)

    out_flat = _make_sc_gather(num_rows)(tab, idx3d)
    x = out_flat.reshape(L, B // 8, 2 * pieces, 8, _P)
    return x.transpose(1, 3, 0, 2, 4).reshape(B, L, 2 * D)


# speedup vs baseline: 18.1617x; 18.1617x over previous
import functools

import jax
import jax.numpy as jnp
from jax import lax
from jax.experimental import pallas as pl
from jax.experimental.pallas import tpu as pltpu
from jax.experimental.pallas import tpu_sc as plsc


def _proj_body(emb_ref, w_ref, b_ref, out_ref):
    out_ref[...] = lax.dot_general(
        emb_ref[...], w_ref[...],
        dimension_numbers=(((1,), (1,)), ((), ())),
        preferred_element_type=jnp.float32,
    ) + b_ref[...]


def _project_table(emb_table, W1, b1):
    V, D = emb_table.shape
    return pl.pallas_call(
        _proj_body,
        out_shape=jax.ShapeDtypeStruct((V, D), jnp.float32),
    )(emb_table, W1, b1.reshape(1, D))


_P = 128
_CHUNK = 64
_SECT_ROWS = 32
_PROJ_ROWS = 4096
_IMG_HALF = 8192


def _make_sc_gather(num_rows, L):
    info = plsc.get_sparse_core_info()
    NC, NS = info.num_cores, info.num_subcores
    NW = NC * NS
    rows_per_w = num_rows // NW
    chunks_per_w = rows_per_w // _CHUNK
    chunks_per_l = chunks_per_w // L
    sections = chunks_per_w // (2 * _SECT_ROWS)
    rows_per_l = num_rows // L
    tab_rows = _PROJ_ROWS + _IMG_HALF
    mesh = plsc.VectorSubcoreMesh(core_axis_name="c", subcore_axis_name="s")

    @functools.partial(
        pl.kernel,
        mesh=mesh,
        out_type=jax.ShapeDtypeStruct((num_rows, _P), jnp.float32),
        scratch_types=[
            pltpu.VMEM((_SECT_ROWS, _P), jnp.int32),
            pltpu.VMEM((_SECT_ROWS, _P), jnp.int32),
            pltpu.VMEM((_CHUNK, _P), jnp.float32),
            pltpu.VMEM((_CHUNK, _P), jnp.float32),
            pltpu.VMEM_SHARED((tab_rows, _P), jnp.float32),
            pltpu.SemaphoreType.DMA,
            pltpu.SemaphoreType.DMA,
            pltpu.SemaphoreType.DMA,
            pltpu.SemaphoreType.DMA,
        ],
    )
    def sc_kernel(tab_hbm, idx_hbm, out_hbm,
                  sb0, sb1, buf0, buf1, tab_sh, gs0, gs1, os0, os1):
        scid = lax.axis_index("c")
        sid = lax.axis_index("s")
        wid = scid * NS + sid
        sbufs = (sb0, sb1)
        bufs = (buf0, buf1)
        gsems = (gs0, gs1)
        osems = (os0, os1)

        pp = _PROJ_ROWS // NS
        ip = _IMG_HALF // NS
        pltpu.sync_copy(tab_hbm.at[pl.ds(pp * sid, pp)],
                        tab_sh.at[pl.ds(pp * sid, pp)])
        pltpu.sync_copy(
            tab_hbm.at[pl.ds(_PROJ_ROWS + _IMG_HALF * scid + ip * sid, ip)],
            tab_sh.at[pl.ds(_PROJ_ROWS + ip * sid, ip)])
        pltpu.sync_copy(idx_hbm.at[wid, pl.ds(0, _SECT_ROWS)], sb0)
        plsc.subcore_barrier()

        def out_slice(j):
            o = (j // chunks_per_l) * rows_per_l \
                + wid * (chunks_per_l * _CHUNK) \
                + (j % chunks_per_l) * _CHUNK
            return out_hbm.at[pl.ds(pl.multiple_of(o, _CHUNK), _CHUNK)]

        def idx_ref(sb, j):
            loc = j % (2 * _SECT_ROWS)
            return sb.at[loc // 2, pl.ds((loc % 2) * _CHUNK, _CHUNK)]

        def fire_gather(sb, j, k):
            return pltpu.async_copy(tab_sh.at[idx_ref(sb, j)], bufs[k],
                                    gsems[k])

        def wait_gather(sb, j, k):
            pltpu.make_async_copy(tab_sh.at[idx_ref(sb, j)], bufs[k],
                                  gsems[k]).wait()

        def fire_write(j, k):
            return pltpu.async_copy(bufs[k], out_slice(j), osems[k])

        def wait_write(j, k):
            pltpu.make_async_copy(bufs[k], out_slice(j), osems[k]).wait()

        fire_gather(sb0, 0, 0)
        fire_gather(sb0, 1, 1)

        for sect in range(sections):
            sb = sbufs[sect & 1]
            sb_next = sbufs[(sect + 1) & 1]
            base = sect * 2 * _SECT_ROWS
            if sect + 1 < sections:
                pltpu.sync_copy(
                    idx_hbm.at[wid, pl.ds((sect + 1) * _SECT_ROWS,
                                          _SECT_ROWS)],
                    sb_next)

            def pair(g, carry, sb=sb, base=base):
                j0 = base + 2 * g
                wait_gather(sb, j0, 0)
                fire_write(j0, 0)
                wait_gather(sb, j0 + 1, 1)
                fire_write(j0 + 1, 1)
                wait_write(j0, 0)
                fire_gather(sb, j0 + 2, 0)
                wait_write(j0 + 1, 1)
                fire_gather(sb, j0 + 3, 1)
                return carry

            lax.fori_loop(0, _SECT_ROWS - 1, pair, 0)

            j0 = base + 2 * (_SECT_ROWS - 1)
            wait_gather(sb, j0, 0)
            fire_write(j0, 0)
            wait_gather(sb, j0 + 1, 1)
            fire_write(j0 + 1, 1)
            wait_write(j0, 0)
            if sect + 1 < sections:
                fire_gather(sb_next, j0 + 2, 0)
            wait_write(j0 + 1, 1)
            if sect + 1 < sections:
                fire_gather(sb_next, j0 + 3, 1)

    return sc_kernel


def kernel(caption_tknID, img_fc, emb_table, W1, b1):
    B, L = caption_tknID.shape
    D = img_fc.shape[1]
    num_tokens = B * L
    pieces = D // _P
    num_rows = num_tokens * 2 * pieces

    proj = _project_table(emb_table, W1, b1)

    tab = jnp.concatenate([
        proj.reshape(emb_table.shape[0] * pieces, _P),
        jnp.zeros((_PROJ_ROWS - emb_table.shape[0] * pieces, _P), jnp.float32),
        img_fc.reshape(B * pieces, _P),
    ], axis=0)

    info = plsc.get_sparse_core_info()
    nw = info.num_cores * info.num_subcores
    ns = info.num_subcores
    tokP = (caption_tknID.astype(jnp.int32)
            .reshape(nw, 16, 8, L).transpose(0, 3, 1, 2))
    cc = jnp.arange(pieces, dtype=jnp.int32).reshape(1, 1, 1, pieces, 1)
    sh = (nw, L, 16, 1, 8)
    cap_loc = ((jax.lax.broadcasted_iota(jnp.int32, sh, 0) % ns) * 128
               + jax.lax.broadcasted_iota(jnp.int32, sh, 2) * 8
               + jax.lax.broadcasted_iota(jnp.int32, sh, 4))
    word_idx = jnp.broadcast_to(tokP[:, :, :, None, :] * pieces + cc,
                                (nw, L, 16, pieces, 8))
    img_idx = jnp.broadcast_to(_PROJ_ROWS + cap_loc * pieces + cc,
                               (nw, L, 16, pieces, 8))
    idx_full = jnp.concatenate([word_idx, img_idx], axis=3)
    idx3d = idx_full.reshape(nw, num_rows // (nw * _P), _P)

    out_flat = _make_sc_gather(num_rows, L)(tab, idx3d)
    x = out_flat.reshape(L, B // 8, 2 * pieces, 8, _P)
    return x.transpose(1, 3, 0, 2, 4).reshape(B, L, 2 * D)


# speedup vs baseline: 19.5101x; 1.0742x over previous
import functools

import jax
import jax.numpy as jnp
from jax import lax
from jax.experimental import pallas as pl
from jax.experimental.pallas import tpu as pltpu
from jax.experimental.pallas import tpu_sc as plsc


def _proj_body(emb_ref, w_ref, b_ref, out_ref):
    out_ref[...] = lax.dot_general(
        emb_ref[...], w_ref[...],
        dimension_numbers=(((1,), (1,)), ((), ())),
        preferred_element_type=jnp.float32,
    ) + b_ref[...]


def _project_table(emb_table, W1, b1):
    V, D = emb_table.shape
    return pl.pallas_call(
        _proj_body,
        out_shape=jax.ShapeDtypeStruct((V, D), jnp.float32),
    )(emb_table, W1, b1.reshape(1, D))


_P = 128
_CHUNK = 64
_SECT_ROWS = 32
_PROJ_ROWS = 4096
_IMG_HALF = 8192


def _make_sc_gather(num_rows, L):
    info = plsc.get_sparse_core_info()
    NC, NS = info.num_cores, info.num_subcores
    NW = NC * NS
    rows_per_w = num_rows // NW
    chunks_per_w = rows_per_w // _CHUNK
    chunks_per_l = chunks_per_w // L
    sections = chunks_per_w // (2 * _SECT_ROWS)
    rows_per_l = num_rows // L
    tab_rows = _PROJ_ROWS + _IMG_HALF
    mesh = plsc.VectorSubcoreMesh(core_axis_name="c", subcore_axis_name="s")

    @functools.partial(
        pl.kernel,
        mesh=mesh,
        out_type=jax.ShapeDtypeStruct((num_rows, _P), jnp.float32),
        scratch_types=[
            pltpu.VMEM((_SECT_ROWS, _P), jnp.int32),
            pltpu.VMEM((_SECT_ROWS, _P), jnp.int32),
            pltpu.VMEM((_CHUNK, _P), jnp.float32),
            pltpu.VMEM((_CHUNK, _P), jnp.float32),
            pltpu.VMEM_SHARED((tab_rows, _P), jnp.float32),
            pltpu.SemaphoreType.DMA,
            pltpu.SemaphoreType.DMA,
            pltpu.SemaphoreType.DMA,
            pltpu.SemaphoreType.DMA,
            pltpu.SemaphoreType.DMA,
        ],
    )
    def sc_kernel(tab_hbm, idx_hbm, out_hbm,
                  sb0, sb1, buf0, buf1, tab_sh, gs0, gs1, os0, os1, ssem):
        scid = lax.axis_index("c")
        sid = lax.axis_index("s")
        wid = scid * NS + sid
        sbufs = (sb0, sb1)
        bufs = (buf0, buf1)
        gsems = (gs0, gs1)
        osems = (os0, os1)

        pp = _PROJ_ROWS // NS
        ip = _IMG_HALF // NS
        pltpu.sync_copy(tab_hbm.at[pl.ds(pp * sid, pp)],
                        tab_sh.at[pl.ds(pp * sid, pp)])
        pltpu.sync_copy(
            tab_hbm.at[pl.ds(_PROJ_ROWS + _IMG_HALF * scid + ip * sid, ip)],
            tab_sh.at[pl.ds(_PROJ_ROWS + ip * sid, ip)])
        pltpu.sync_copy(idx_hbm.at[wid, pl.ds(0, _SECT_ROWS)], sb0)
        plsc.subcore_barrier()

        def out_slice(j):
            o = (j // chunks_per_l) * rows_per_l \
                + wid * (chunks_per_l * _CHUNK) \
                + (j % chunks_per_l) * _CHUNK
            return out_hbm.at[pl.ds(pl.multiple_of(o, _CHUNK), _CHUNK)]

        def idx_ref(sb, j):
            loc = j % (2 * _SECT_ROWS)
            return sb.at[loc // 2, pl.ds((loc % 2) * _CHUNK, _CHUNK)]

        def fire_gather(sb, j, k):
            return pltpu.async_copy(tab_sh.at[idx_ref(sb, j)], bufs[k],
                                    gsems[k])

        def wait_gather(sb, j, k):
            pltpu.make_async_copy(tab_sh.at[idx_ref(sb, j)], bufs[k],
                                  gsems[k]).wait()

        def fire_write(j, k):
            return pltpu.async_copy(bufs[k], out_slice(j), osems[k])

        def wait_write(j, k):
            pltpu.make_async_copy(bufs[k], out_slice(j), osems[k]).wait()

        fire_gather(sb0, 0, 0)
        fire_gather(sb0, 1, 1)

        def stage_src(s):
            return idx_hbm.at[wid, pl.ds(s * _SECT_ROWS, _SECT_ROWS)]

        for sect in range(sections):
            sb = sbufs[sect & 1]
            sb_next = sbufs[(sect + 1) & 1]
            base = sect * 2 * _SECT_ROWS
            if sect + 1 < sections:
                pltpu.async_copy(stage_src(sect + 1), sb_next, ssem)

            def pair(g, carry, sb=sb, base=base):
                j0 = base + 2 * g
                wait_gather(sb, j0, 0)
                fire_write(j0, 0)
                wait_gather(sb, j0 + 1, 1)
                fire_write(j0 + 1, 1)
                wait_write(j0, 0)
                fire_gather(sb, j0 + 2, 0)
                wait_write(j0 + 1, 1)
                fire_gather(sb, j0 + 3, 1)
                return carry

            lax.fori_loop(0, _SECT_ROWS - 1, pair, 0)

            j0 = base + 2 * (_SECT_ROWS - 1)
            wait_gather(sb, j0, 0)
            fire_write(j0, 0)
            wait_gather(sb, j0 + 1, 1)
            fire_write(j0 + 1, 1)
            wait_write(j0, 0)
            if sect + 1 < sections:
                pltpu.make_async_copy(stage_src(sect + 1), sb_next,
                                      ssem).wait()
                fire_gather(sb_next, j0 + 2, 0)
            wait_write(j0 + 1, 1)
            if sect + 1 < sections:
                fire_gather(sb_next, j0 + 3, 1)

    return sc_kernel


def kernel(caption_tknID, img_fc, emb_table, W1, b1):
    B, L = caption_tknID.shape
    D = img_fc.shape[1]
    num_tokens = B * L
    pieces = D // _P
    num_rows = num_tokens * 2 * pieces

    proj = _project_table(emb_table, W1, b1)

    tab = jnp.concatenate([
        proj.reshape(emb_table.shape[0] * pieces, _P),
        jnp.zeros((_PROJ_ROWS - emb_table.shape[0] * pieces, _P), jnp.float32),
        img_fc.reshape(B * pieces, _P),
    ], axis=0)

    info = plsc.get_sparse_core_info()
    nw = info.num_cores * info.num_subcores
    ns = info.num_subcores
    tokP = (caption_tknID.astype(jnp.int32)
            .reshape(nw, 16, 8, L).transpose(0, 3, 1, 2))
    cc = jnp.arange(pieces, dtype=jnp.int32).reshape(1, 1, 1, pieces, 1)
    sh = (nw, L, 16, 1, 8)
    cap_loc = ((jax.lax.broadcasted_iota(jnp.int32, sh, 0) % ns) * 128
               + jax.lax.broadcasted_iota(jnp.int32, sh, 2) * 8
               + jax.lax.broadcasted_iota(jnp.int32, sh, 4))
    word_idx = jnp.broadcast_to(tokP[:, :, :, None, :] * pieces + cc,
                                (nw, L, 16, pieces, 8))
    img_idx = jnp.broadcast_to(_PROJ_ROWS + cap_loc * pieces + cc,
                               (nw, L, 16, pieces, 8))
    idx_full = jnp.concatenate([word_idx, img_idx], axis=3)
    idx3d = idx_full.reshape(nw, num_rows // (nw * _P), _P)

    out_flat = _make_sc_gather(num_rows, L)(tab, idx3d)
    x = out_flat.reshape(L, B // 8, 2 * pieces, 8, _P)
    return x.transpose(1, 3, 0, 2, 4).reshape(B, L, 2 * D)
